# baseline (speedup 1.0000x reference)
import functools

import jax
import jax.numpy as jnp
from jax import lax
from jax.experimental import pallas as pl
from jax.experimental.pallas import tpu as pltpu
from jax.experimental.pallas import tpu_sc as plsc

B, R, W_CNT, D = 4096, 100, 200, 128
HALF = B // 2

NC, NS, L = 2, 16, 16
TILE = 128
COLS_PW = TILE // 2
G_PW = COLS_PW // L
PAIRS_PC = NS // 2
RP = 104
UNROLL = 8


def _sc_counts_body(half, wt_hbm, m_hbm, out_hbm, wt_v, m_v, tab_v, shr_v,
                    idx_v, sem_wt, sem_m):
    c = lax.axis_index("c")
    s = lax.axis_index("s")
    pair = s // 2
    q = s % 2
    gtile = c * PAIRS_PC + pair
    col0 = gtile * TILE
    src0 = half * HALF + col0
    cp_wt = pltpu.async_copy(wt_hbm.at[:, pl.ds(src0, TILE)], wt_v, sem_wt)
    cp_m = pltpu.async_copy(m_hbm.at[:, pl.ds(src0, TILE)], m_v, sem_m)

    zeros = jnp.zeros((L,), jnp.float32)
    lane = lax.iota(jnp.int32, L)

    def zbody(i, _):
        for u in range(TILE // L):
            tab_v[i, pl.ds(u * L, L)] = zeros
        return 0

    lax.fori_loop(0, RP, zbody, 0)
    for off in (0, 16, 32, 48, 64, 80, 88):
        idx_v[pl.ds(off, L)] = pair * RP + off + lane

    @pl.when(q == 0)
    def _():
        pltpu.sync_copy(tab_v, shr_v.at[pl.ds(pair * RP, RP)])

    plsc.subcore_barrier()
    cp_wt.wait()
    cp_m.wait()

    qoff = q * COLS_PW
    for g in range(G_PW):
        col_base = qoff + g * L + lane

        def jbody(j, _, col_base=col_base, g=g):
            for u in range(UNROLL):
                jj = j * UNROLL + u
                idx_r = wt_v[jj, pl.ds(qoff + g * L, L)]
                val = m_v[jj, pl.ds(qoff + g * L, L)]
                plsc.addupdate_scatter(tab_v, [idx_r, col_base], val)
            return 0

        lax.fori_loop(0, W_CNT // UNROLL, jbody, 0)

    pltpu.sync_copy(tab_v, shr_v.at[idx_v], add=True)
    plsc.subcore_barrier()

    @pl.when(q == 0)
    def _():
        pltpu.sync_copy(shr_v.at[pl.ds(pair * RP, R)],
                        out_hbm.at[:, pl.ds(col0, TILE)])


def _make_sc_counts(half):
    return pl.kernel(
        functools.partial(_sc_counts_body, half),
        out_type=jax.ShapeDtypeStruct((R, HALF), jnp.float32),
        mesh=plsc.VectorSubcoreMesh(core_axis_name="c", subcore_axis_name="s"),
        scratch_types=[
            pltpu.VMEM((W_CNT, TILE), jnp.int32),
            pltpu.VMEM((W_CNT, TILE), jnp.float32),
            pltpu.VMEM((RP, TILE), jnp.float32),
            pltpu.VMEM_SHARED((PAIRS_PC * RP, TILE), jnp.float32),
            pltpu.VMEM((RP,), jnp.int32),
            pltpu.SemaphoreType.DMA,
            pltpu.SemaphoreType.DMA,
        ],
        compiler_params=pltpu.CompilerParams(needs_layout_passes=False),
    )


_sc_counts = [_make_sc_counts(0), _make_sc_counts(1)]

BB = 256
JC = 8


def _tc_body(counts_ref, enc_ref, w_ref, out_ref):
    pooled = jnp.zeros((BB, D), jnp.float32)
    for c in range(pl.cdiv(R, JC)):
        lo, hi = c * JC, min((c + 1) * JC, R)
        encc = enc_ref[lo:hi, :, :]
        ccc = counts_ref[lo:hi, :]
        pooled = pooled + (encc * ccc[:, :, None]).sum(axis=0)

    out_ref[...] = jnp.dot(pooled, w_ref[...].T,
                           preferred_element_type=jnp.float32)


def _tc_half(half, counts_t, enc_t, W):
    blk0 = half * (HALF // BB)
    return pl.pallas_call(
        _tc_body,
        grid=(HALF // BB,),
        in_specs=[
            pl.BlockSpec((R, BB), lambda i: (0, i)),
            pl.BlockSpec((R, BB, D), lambda i, blk0=blk0: (0, i + blk0, 0)),
            pl.BlockSpec((D, D), lambda i: (0, 0)),
        ],
        out_specs=pl.BlockSpec((BB, D), lambda i: (i, 0)),
        out_shape=jax.ShapeDtypeStruct((HALF, D), jnp.float32),
    )(counts_t, enc_t, W)


@jax.jit
def kernel(encoded_row, wafer_types, loadport_mask, W):
    wt_t = wafer_types.astype(jnp.int32).T
    m_t = loadport_mask.astype(jnp.float32).T
    enc_t = encoded_row.transpose(1, 0, 2)

    outs = []
    for half in (0, 1):
        counts_t = _sc_counts[half](wt_t, m_t)
        outs.append(_tc_half(half, counts_t, enc_t, W))
    return jnp.concatenate(outs, axis=0)


# speedup vs baseline: 1.1244x; 1.1244x over previous
import functools

import jax
import jax.numpy as jnp
from jax import lax
from jax.experimental import pallas as pl
from jax.experimental.pallas import tpu as pltpu
from jax.experimental.pallas import tpu_sc as plsc

B, R, W_CNT, D = 4096, 100, 200, 128

NC, NS, L = 2, 16, 16
NW = NC * NS
ROWS_PW = B // NW
G_PW = ROWS_PW // L
UNROLL = 8


def _sc_counts_body(wt_hbm, m_hbm, out_hbm, wt_v, m_v, tab_v, sem_wt, sem_m):
    wid = lax.axis_index("s") * NC + lax.axis_index("c")
    col0 = wid * ROWS_PW
    cp_wt = pltpu.async_copy(wt_hbm.at[:, pl.ds(col0, ROWS_PW)], wt_v, sem_wt)
    cp_m = pltpu.async_copy(m_hbm.at[:, pl.ds(col0, ROWS_PW)], m_v, sem_m)

    zeros = jnp.zeros((L,), jnp.float32)

    def zbody(i, _):
        for u in range(G_PW):
            tab_v[i, pl.ds(u * L, L)] = zeros
        return 0

    lax.fori_loop(0, R, zbody, 0)
    cp_wt.wait()
    cp_m.wait()

    lane = lax.iota(jnp.int32, L)
    for g in range(G_PW):
        col_base = g * L + lane

        def _make_jbody(g, col_base):
            def jbody(j):
                idx_r = wt_v[j, pl.ds(g * L, L)]
                val = m_v[j, pl.ds(g * L, L)]
                plsc.addupdate_scatter(tab_v, [idx_r, col_base], val)
            return jbody

        plsc.parallel_loop(0, W_CNT, 1, unroll=UNROLL)(
            _make_jbody(g, col_base))

    pltpu.sync_copy(tab_v, out_hbm.at[:, pl.ds(col0, ROWS_PW)])


_sc_counts = pl.kernel(
    _sc_counts_body,
    out_type=jax.ShapeDtypeStruct((R, B), jnp.float32),
    mesh=plsc.VectorSubcoreMesh(core_axis_name="c", subcore_axis_name="s"),
    scratch_types=[
        pltpu.VMEM((W_CNT, ROWS_PW), jnp.int32),
        pltpu.VMEM((W_CNT, ROWS_PW), jnp.float32),
        pltpu.VMEM((R, ROWS_PW), jnp.float32),
        pltpu.SemaphoreType.DMA,
        pltpu.SemaphoreType.DMA,
    ],
    compiler_params=pltpu.CompilerParams(needs_layout_passes=False),
)

BB = 256
JC = 8


def _tc_body(counts_ref, enc_ref, w_ref, out_ref):
    pooled = jnp.zeros((BB, D), jnp.float32)
    for c in range(pl.cdiv(R, JC)):
        lo, hi = c * JC, min((c + 1) * JC, R)
        encc = enc_ref[lo:hi, :, :]
        ccc = counts_ref[lo:hi, :]
        pooled = pooled + (encc * ccc[:, :, None]).sum(axis=0)

    out_ref[...] = jnp.dot(pooled, w_ref[...].T,
                           preferred_element_type=jnp.float32)


@jax.jit
def kernel(encoded_row, wafer_types, loadport_mask, W):
    wt_t = wafer_types.astype(jnp.int32).T
    m_t = loadport_mask.astype(jnp.float32).T
    counts_t = _sc_counts(wt_t, m_t)
    enc_t = encoded_row.transpose(1, 0, 2)

    grid = (B // BB,)
    return pl.pallas_call(
        _tc_body,
        grid=grid,
        in_specs=[
            pl.BlockSpec((R, BB), lambda i: (0, i)),
            pl.BlockSpec((R, BB, D), lambda i: (0, i, 0)),
            pl.BlockSpec((D, D), lambda i: (0, 0)),
        ],
        out_specs=pl.BlockSpec((BB, D), lambda i: (i, 0)),
        out_shape=jax.ShapeDtypeStruct((B, D), jnp.float32),
    )(counts_t, enc_t, W)
